# baseline (speedup 1.0000x reference)
import jax
import jax.numpy as jnp
from jax.experimental import pallas as pl
from jax.experimental.pallas import tpu as pltpu

_CHUNKS = (200, 2400, 2400, 2400, 2400, 200)


def _copy_kernel(x_hbm, o_hbm, buf, in_sems, out_sems):
    offs = [sum(_CHUNKS[:c]) for c in range(len(_CHUNKS))]

    def in_copy(c):
        sl = pl.ds(offs[c], _CHUNKS[c])
        return pltpu.make_async_copy(x_hbm.at[sl, :], buf.at[sl, :],
                                     in_sems.at[c])

    def out_copy(c):
        sl = pl.ds(offs[c], _CHUNKS[c])
        return pltpu.make_async_copy(buf.at[sl, :], o_hbm.at[sl, :],
                                     out_sems.at[c])

    for c in range(len(_CHUNKS)):
        in_copy(c).start()
    for c in range(len(_CHUNKS)):
        in_copy(c).wait()
        out_copy(c).start()
    for c in range(len(_CHUNKS)):
        out_copy(c).wait()


def kernel(x, edge_index):
    n, d = x.shape
    n_chunks = len(_CHUNKS)
    return pl.pallas_call(
        _copy_kernel,
        in_specs=[pl.BlockSpec(memory_space=pl.ANY)],
        out_specs=pl.BlockSpec(memory_space=pl.ANY),
        out_shape=jax.ShapeDtypeStruct((n, d), x.dtype),
        scratch_shapes=[
            pltpu.VMEM((n, d), x.dtype),
            pltpu.SemaphoreType.DMA((n_chunks,)),
            pltpu.SemaphoreType.DMA((n_chunks,)),
        ],
    )(x)


# speedup vs baseline: 1.0050x; 1.0050x over previous
import jax
import jax.numpy as jnp
from jax.experimental import pallas as pl
from jax.experimental.pallas import tpu as pltpu

_CHUNKS = (400, 2400, 2400, 2400, 2400)


def _copy_kernel(x_hbm, o_hbm, buf, in_sems, out_sems):
    offs = [sum(_CHUNKS[:c]) for c in range(len(_CHUNKS))]

    def in_copy(c):
        sl = pl.ds(offs[c], _CHUNKS[c])
        return pltpu.make_async_copy(x_hbm.at[sl, :], buf.at[sl, :],
                                     in_sems.at[c])

    def out_copy(c):
        sl = pl.ds(offs[c], _CHUNKS[c])
        return pltpu.make_async_copy(buf.at[sl, :], o_hbm.at[sl, :],
                                     out_sems.at[c])

    for c in range(len(_CHUNKS)):
        in_copy(c).start()
    for c in range(len(_CHUNKS)):
        in_copy(c).wait()
        out_copy(c).start()
    for c in range(len(_CHUNKS)):
        out_copy(c).wait()


def kernel(x, edge_index):
    n, d = x.shape
    n_chunks = len(_CHUNKS)
    return pl.pallas_call(
        _copy_kernel,
        in_specs=[pl.BlockSpec(memory_space=pl.ANY)],
        out_specs=pl.BlockSpec(memory_space=pl.ANY),
        out_shape=jax.ShapeDtypeStruct((n, d), x.dtype),
        scratch_shapes=[
            pltpu.VMEM((n, d), x.dtype),
            pltpu.SemaphoreType.DMA((n_chunks,)),
            pltpu.SemaphoreType.DMA((n_chunks,)),
        ],
    )(x)


# speedup vs baseline: 1.0604x; 1.0552x over previous
import jax
import jax.numpy as jnp
from jax.experimental import pallas as pl
from jax.experimental.pallas import tpu as pltpu

_CHUNKS = (400, 800, 2400, 2400, 2400, 1200, 400)


def _copy_kernel(x_hbm, o_hbm, buf, in_sems, out_sems):
    offs = [sum(_CHUNKS[:c]) for c in range(len(_CHUNKS))]

    def in_copy(c):
        sl = pl.ds(offs[c], _CHUNKS[c])
        return pltpu.make_async_copy(x_hbm.at[sl, :], buf.at[sl, :],
                                     in_sems.at[c])

    def out_copy(c):
        sl = pl.ds(offs[c], _CHUNKS[c])
        return pltpu.make_async_copy(buf.at[sl, :], o_hbm.at[sl, :],
                                     out_sems.at[c])

    for c in range(len(_CHUNKS)):
        in_copy(c).start()
    for c in range(len(_CHUNKS)):
        in_copy(c).wait()
        out_copy(c).start()
    for c in range(len(_CHUNKS)):
        out_copy(c).wait()


def kernel(x, edge_index):
    n, d = x.shape
    n_chunks = len(_CHUNKS)
    return pl.pallas_call(
        _copy_kernel,
        in_specs=[pl.BlockSpec(memory_space=pl.ANY)],
        out_specs=pl.BlockSpec(memory_space=pl.ANY),
        out_shape=jax.ShapeDtypeStruct((n, d), x.dtype),
        scratch_shapes=[
            pltpu.VMEM((n, d), x.dtype),
            pltpu.SemaphoreType.DMA((n_chunks,)),
            pltpu.SemaphoreType.DMA((n_chunks,)),
        ],
    )(x)
